# baseline (device time: 27805 ns/iter reference)
import jax
import jax.numpy as jnp
from jax import lax
from jax.experimental import pallas as pl
from jax.experimental.pallas import tpu as pltpu

N_DEV = 4
HC = 128


def kernel(A, B):
    m, k = A.shape
    _, n = B.shape
    h2 = m // 4

    def body(a_hbm, b_hbm, out_ref, a_ref, b_ref, acc, send, recv,
             send_sems, recv_sems, hbm_sems, in_sems):
        a_cp = pltpu.make_async_copy(a_hbm, a_ref, in_sems.at[0])
        b_cp = pltpu.make_async_copy(b_hbm, b_ref, in_sems.at[1])
        a_cp.start()
        b_cp.start()
        my = lax.axis_index("i")
        bit0 = jnp.bitwise_and(my, 1)
        bit1 = jnp.bitwise_and(jnp.right_shift(my, 1), 1)
        p1 = jnp.bitwise_xor(my, 1)
        p2 = jnp.bitwise_xor(my, 3)

        k1 = jnp.bitwise_xor(bit0, bit1)
        k1p = bit1

        a_ok = k1 * h2
        a_os = (1 - k1) * h2
        b_ok = 2 * h2 + k1p * h2
        b_os = 2 * h2 + (1 - k1p) * h2

        cfg = [
            ((p1, p2, p1), a_ok, a_os),
            ((p2, p1, p2), b_ok, b_os),
        ]

        def slot(stage, h, c):
            return stage * 4 + h * 2 + c

        barrier_sem = pltpu.get_barrier_semaphore()
        for nbr in [p1, p2]:
            pl.semaphore_signal(
                barrier_sem, inc=1,
                device_id=(nbr,), device_id_type=pl.DeviceIdType.MESH,
            )

        a_cp.wait()
        b_cp.wait()
        b_bf = b_ref[...].astype(jnp.bfloat16)

        def mm_chunk(off):
            part = jnp.dot(
                a_ref[pl.ds(off, HC), :].astype(jnp.bfloat16), b_bf,
                preferred_element_type=jnp.float32,
            )
            acc[pl.ds(off, HC), :] = part
            return part

        def exchange(idx, dev):
            return pltpu.make_async_remote_copy(
                src_ref=send.at[idx], dst_ref=recv.at[idx],
                send_sem=send_sems.at[idx], recv_sem=recv_sems.at[idx],
                device_id=(dev,), device_id_type=pl.DeviceIdType.MESH,
            )

        hbm_copies = []

        def store_hbm(off, sem_i):
            cp = pltpu.make_async_copy(
                acc.at[pl.ds(off, HC), :],
                out_ref.at[pl.ds(off, HC), :],
                hbm_sems.at[sem_i],
            )
            cp.start()
            hbm_copies.append(cp)

        order = [(0, 0), (1, 0), (0, 1), (1, 1)]
        rd = {}

        for idx, (h, c) in enumerate(order):
            (parts, _, os_) = cfg[h]
            i = slot(0, h, c)
            send[i] = mm_chunk(os_ + c * HC).astype(jnp.bfloat16)
            if idx == 0:
                pl.semaphore_wait(barrier_sem, 2)
            rd[(0, h, c)] = exchange(i, parts[0])
            rd[(0, h, c)].start()

        for h, c in order:
            mm_chunk(cfg[h][1] + c * HC)

        for h, c in order:
            (parts, ok, _) = cfg[h]
            rd[(0, h, c)].wait_recv()
            off = ok + c * HC
            val = acc[pl.ds(off, HC), :] + recv[
                slot(0, h, c)].astype(jnp.float32)
            acc[pl.ds(off, HC), :] = val
            i = slot(1, h, c)
            send[i] = val.astype(jnp.bfloat16)
            rd[(1, h, c)] = exchange(i, parts[1])
            rd[(1, h, c)].start()

        for h, c in order:
            (parts, ok, _) = cfg[h]
            rd[(1, h, c)].wait_recv()
            off = ok + c * HC
            val = acc[pl.ds(off, HC), :] + recv[
                slot(1, h, c)].astype(jnp.float32)
            acc[pl.ds(off, HC), :] = val
            i = slot(2, h, c)
            send[i] = val.astype(jnp.bfloat16)
            rd[(2, h, c)] = exchange(i, parts[2])
            rd[(2, h, c)].start()
            store_hbm(off, slot(0, h, c))

        for h, c in order:
            (_, _, os_) = cfg[h]
            rd[(2, h, c)].wait_recv()
            off = os_ + c * HC
            acc[pl.ds(off, HC), :] = recv[
                slot(2, h, c)].astype(jnp.float32)
            store_hbm(off, 4 + slot(0, h, c))

        for cp in hbm_copies:
            cp.wait()
        for r in rd.values():
            r.wait_send()

    return pl.pallas_call(
        body,
        out_shape=jax.ShapeDtypeStruct((m, n), jnp.float32),
        in_specs=[
            pl.BlockSpec(memory_space=pltpu.MemorySpace.HBM),
            pl.BlockSpec(memory_space=pltpu.MemorySpace.HBM),
        ],
        out_specs=pl.BlockSpec(memory_space=pltpu.MemorySpace.HBM),
        scratch_shapes=[
            pltpu.VMEM((m, k), jnp.float32),
            pltpu.VMEM((k, n), jnp.float32),
            pltpu.VMEM((m, n), jnp.float32),
            pltpu.VMEM((12, HC, n), jnp.bfloat16),
            pltpu.VMEM((12, HC, n), jnp.bfloat16),
            pltpu.SemaphoreType.DMA((12,)),
            pltpu.SemaphoreType.DMA((12,)),
            pltpu.SemaphoreType.DMA((8,)),
            pltpu.SemaphoreType.DMA((2,)),
        ],
        compiler_params=pltpu.CompilerParams(collective_id=0),
    )(A, B)


# device time: 27346 ns/iter; 1.0168x vs baseline; 1.0168x over previous
import jax
import jax.numpy as jnp
from jax import lax
from jax.experimental import pallas as pl
from jax.experimental.pallas import tpu as pltpu

N_DEV = 4
HC = 128


def kernel(A, B):
    m, k = A.shape
    _, n = B.shape
    h2 = m // 4

    def body(a_ref, b_ref, out_ref, send, recv, send_sems, recv_sems):
        my = lax.axis_index("i")
        bit0 = jnp.bitwise_and(my, 1)
        bit1 = jnp.bitwise_and(jnp.right_shift(my, 1), 1)
        p1 = jnp.bitwise_xor(my, 1)
        p2 = jnp.bitwise_xor(my, 3)

        k1 = jnp.bitwise_xor(bit0, bit1)
        k1p = bit1

        a_ok = k1 * h2
        a_os = (1 - k1) * h2
        b_ok = 2 * h2 + k1p * h2
        b_os = 2 * h2 + (1 - k1p) * h2

        cfg = [
            ((p1, p2, p1), a_ok, a_os),
            ((p2, p1, p2), b_ok, b_os),
        ]

        def slot(stage, h, c):
            return stage * 4 + h * 2 + c

        barrier_sem = pltpu.get_barrier_semaphore()
        for nbr in [p1, p2]:
            pl.semaphore_signal(
                barrier_sem, inc=1,
                device_id=(nbr,), device_id_type=pl.DeviceIdType.MESH,
            )

        b_bf = b_ref[...].astype(jnp.bfloat16)

        def mm_chunk(off):
            part = jnp.dot(
                a_ref[pl.ds(off, HC), :].astype(jnp.bfloat16), b_bf,
                preferred_element_type=jnp.float32,
            )
            out_ref[pl.ds(off, HC), :] = part
            return part

        def exchange(idx, dev):
            return pltpu.make_async_remote_copy(
                src_ref=send.at[idx], dst_ref=recv.at[idx],
                send_sem=send_sems.at[idx], recv_sem=recv_sems.at[idx],
                device_id=(dev,), device_id_type=pl.DeviceIdType.MESH,
            )

        order = [(0, 0), (1, 0), (0, 1), (1, 1)]
        rd = {}

        for h, c in order:
            (parts, _, os_) = cfg[h]
            i = slot(0, h, c)
            send[i] = mm_chunk(os_ + c * HC).astype(jnp.bfloat16)
            if (h, c) == (0, 0):
                pl.semaphore_wait(barrier_sem, 2)
            rd[(0, h, c)] = exchange(i, parts[0])
            rd[(0, h, c)].start()

        for h, c in order:
            mm_chunk(cfg[h][1] + c * HC)

        for h, c in order:
            (parts, ok, _) = cfg[h]
            rd[(0, h, c)].wait_recv()
            off = ok + c * HC
            val = out_ref[pl.ds(off, HC), :] + recv[
                slot(0, h, c)].astype(jnp.float32)
            out_ref[pl.ds(off, HC), :] = val
            i = slot(1, h, c)
            send[i] = val.astype(jnp.bfloat16)
            rd[(1, h, c)] = exchange(i, parts[1])
            rd[(1, h, c)].start()

        for h, c in order:
            (parts, ok, _) = cfg[h]
            rd[(1, h, c)].wait_recv()
            off = ok + c * HC
            val = out_ref[pl.ds(off, HC), :] + recv[
                slot(1, h, c)].astype(jnp.float32)
            out_ref[pl.ds(off, HC), :] = val
            i = slot(2, h, c)
            send[i] = val.astype(jnp.bfloat16)
            rd[(2, h, c)] = exchange(i, parts[2])
            rd[(2, h, c)].start()

        for h, c in order:
            (_, _, os_) = cfg[h]
            rd[(2, h, c)].wait_recv()
            out_ref[pl.ds(os_ + c * HC, HC), :] = recv[
                slot(2, h, c)].astype(jnp.float32)

        for r in rd.values():
            r.wait_send()

    return pl.pallas_call(
        body,
        out_shape=jax.ShapeDtypeStruct((m, n), jnp.float32),
        in_specs=[
            pl.BlockSpec(memory_space=pltpu.VMEM),
            pl.BlockSpec(memory_space=pltpu.VMEM),
        ],
        out_specs=pl.BlockSpec(memory_space=pltpu.VMEM),
        scratch_shapes=[
            pltpu.VMEM((12, HC, n), jnp.bfloat16),
            pltpu.VMEM((12, HC, n), jnp.bfloat16),
            pltpu.SemaphoreType.DMA((12,)),
            pltpu.SemaphoreType.DMA((12,)),
        ],
        compiler_params=pltpu.CompilerParams(collective_id=0),
    )(A, B)
